# Initial kernel scaffold; baseline (speedup 1.0000x reference)
import functools

import jax
import jax.numpy as jnp
from jax import lax
from jax.experimental import pallas as pl
from jax.experimental.pallas import tpu as pltpu
from jax.experimental.pallas import tpu_sc as plsc

N = 100000
N_CONN = 80000
N_UNC = N - N_CONN
E = 6400000
C = 1000000

NC = 2
NS = 16
NW = NC * NS

LANE = 128
KSUB = 8
BLOCK = LANE * KSUB


E_BLOCKS = E // BLOCK
E_PER, E_EXTRA = divmod(E_BLOCKS, NW)
ROWS_PER_TILE = N_CONN // NS


def _agg_body(F, h_hbm, e0_hbm, e1_hbm, zero_hbm, out_hbm,
              idx0_v, idx1_v, rows_v, gsem):
  cid = lax.axis_index("c")
  sid = lax.axis_index("s")
  wid = cid * NS + sid

  def run(agg_sh):
    zslc = pl.ds(sid * ROWS_PER_TILE, ROWS_PER_TILE)
    pltpu.sync_copy(zero_hbm.at[zslc], agg_sh.at[zslc])
    plsc.subcore_barrier()

    start = wid * E_PER + jnp.minimum(wid, E_EXTRA)
    nblk = E_PER + jnp.where(wid < E_EXTRA, 1, 0)

    def blk(b, _):
      rowbase = (start + b) * KSUB
      pltpu.sync_copy(e0_hbm.at[pl.ds(rowbase, KSUB)], idx0_v)
      pltpu.sync_copy(e1_hbm.at[pl.ds(rowbase, KSUB)], idx1_v)
      for j in range(KSUB):
        pltpu.async_copy(h_hbm.at[idx0_v.at[j]], rows_v.at[j], gsem)
      for j in range(KSUB):
        pltpu.make_async_copy(h_hbm.at[idx0_v.at[j]], rows_v.at[j],
                              gsem).wait()
        pltpu.sync_copy(rows_v.at[j], agg_sh.at[idx1_v.at[j]], add=True)
      return 0

    lax.fori_loop(0, nblk, blk, 0)
    plsc.subcore_barrier()

    oslc = pl.ds(cid * N_CONN + sid * ROWS_PER_TILE, ROWS_PER_TILE)
    pltpu.sync_copy(agg_sh.at[zslc], out_hbm.at[oslc])

  pl.run_scoped(run, pltpu.VMEM_SHARED((N_CONN, F), jnp.float32))


def _make_agg(F):
  mesh = plsc.VectorSubcoreMesh(core_axis_name="c", subcore_axis_name="s",
                                num_cores=NC, num_subcores=NS)
  return pl.kernel(
      functools.partial(_agg_body, F),
      out_type=jax.ShapeDtypeStruct((NC * N_CONN, F), jnp.float32),
      mesh=mesh,
      scratch_types=[
          pltpu.VMEM((KSUB, LANE), jnp.int32),
          pltpu.VMEM((KSUB, LANE), jnp.int32),
          pltpu.VMEM((KSUB, LANE, F), jnp.float32),
          pltpu.SemaphoreType.DMA,
      ],
  )


CP_BLOCKS = -(-C // BLOCK) + 1
CP = CP_BLOCKS * BLOCK
C_PER, C_EXTRA = divmod(CP_BLOCKS, NW)


def _gather_body(h2_hbm, xu_hbm, c0_hbm, c1_hbm, g1_hbm, g2_hbm,
                 idx0_v, idx1_v, rows1_v, rows2_v, gsem):
  cid = lax.axis_index("c")
  sid = lax.axis_index("s")
  wid = cid * NS + sid
  start = wid * C_PER + jnp.minimum(wid, C_EXTRA)
  nblk = C_PER + jnp.where(wid < C_EXTRA, 1, 0)

  def blk(b, _):
    base = (start + b) * BLOCK
    pltpu.sync_copy(c0_hbm.at[pl.ds(base, BLOCK)], idx0_v)
    pltpu.sync_copy(c1_hbm.at[pl.ds(base, BLOCK)], idx1_v)
    for j in range(KSUB):
      lslc = pl.ds(j * LANE, LANE)
      pltpu.async_copy(h2_hbm.at[idx0_v.at[lslc]], rows1_v.at[j], gsem)
      pltpu.async_copy(xu_hbm.at[idx1_v.at[lslc]], rows2_v.at[j], gsem)
    for j in range(KSUB):
      lslc = pl.ds(j * LANE, LANE)
      pltpu.make_async_copy(h2_hbm.at[idx0_v.at[lslc]], rows1_v.at[j],
                            gsem).wait()
      pltpu.make_async_copy(xu_hbm.at[idx1_v.at[lslc]], rows2_v.at[j],
                            gsem).wait()
    rowblk = (start + b) * KSUB
    pltpu.sync_copy(rows1_v, g1_hbm.at[pl.ds(rowblk, KSUB)])
    pltpu.sync_copy(rows2_v, g2_hbm.at[pl.ds(rowblk, KSUB)])
    return 0

  lax.fori_loop(0, nblk, blk, 0)


def _make_gather():
  mesh = plsc.VectorSubcoreMesh(core_axis_name="c", subcore_axis_name="s",
                                num_cores=NC, num_subcores=NS)
  return pl.kernel(
      _gather_body,
      out_type=(
          jax.ShapeDtypeStruct((CP_BLOCKS * KSUB, LANE, 16), jnp.float32),
          jax.ShapeDtypeStruct((CP_BLOCKS * KSUB, LANE, 16), jnp.float32),
      ),
      mesh=mesh,
      scratch_types=[
          pltpu.VMEM((BLOCK,), jnp.int32),
          pltpu.VMEM((BLOCK,), jnp.int32),
          pltpu.VMEM((KSUB, LANE, 16), jnp.float32),
          pltpu.VMEM((KSUB, LANE, 16), jnp.float32),
          pltpu.SemaphoreType.DMA,
      ],
  )


def _layer_body(agg0_ref, agg1_ref, h_ref, wr_ref, br_ref, wt_ref, out_ref):
  a = agg0_ref[...] + agg1_ref[...]
  z = (jnp.dot(a, wr_ref[...], preferred_element_type=jnp.float32)
       + jnp.dot(h_ref[...], wt_ref[...], preferred_element_type=jnp.float32)
       + br_ref[...])
  out_ref[...] = jnp.maximum(z, 0.0)


def _tc_layer(aggflat, h, wrT, br, wtT, fin, fout):
  m = h.shape[0]
  blk = 8000
  grid = m // blk
  nb = m // blk
  return pl.pallas_call(
      _layer_body,
      grid=(grid,),
      in_specs=[
          pl.BlockSpec((blk, fin), lambda i: (i, 0)),
          pl.BlockSpec((blk, fin), lambda i, _nb=nb: (i + _nb, 0)),
          pl.BlockSpec((blk, fin), lambda i: (i, 0)),
          pl.BlockSpec((fin, fout), lambda i: (0, 0)),
          pl.BlockSpec((1, fout), lambda i: (0, 0)),
          pl.BlockSpec((fin, fout), lambda i: (0, 0)),
      ],
      out_specs=pl.BlockSpec((blk, fout), lambda i: (i, 0)),
      out_shape=jax.ShapeDtypeStruct((m, fout), jnp.float32),
  )(aggflat, aggflat, h, wrT, br, wtT)


def _xu_body(x_ref, w_ref, b_ref, out_ref):
  out_ref[...] = (jnp.dot(x_ref[...], w_ref[...],
                          preferred_element_type=jnp.float32) + b_ref[...])


def _tc_xu(xu_in, wuT, bu):
  m = xu_in.shape[0]
  blk = 4000
  return pl.pallas_call(
      _xu_body,
      grid=(m // blk,),
      in_specs=[
          pl.BlockSpec((blk, 2), lambda i: (i, 0)),
          pl.BlockSpec((2, 16), lambda i: (0, 0)),
          pl.BlockSpec((1, 16), lambda i: (0, 0)),
      ],
      out_specs=pl.BlockSpec((blk, 16), lambda i: (i, 0)),
      out_shape=jax.ShapeDtypeStruct((m, 16), jnp.float32),
  )(xu_in, wuT, bu)


MLP_BLK = 2048


def _mlp_body(g1_ref, g2_ref, wa_ref, wb_ref, b1_ref, w2_ref, b2_ref,
              out_ref):
  hid = (jnp.dot(g1_ref[...], wa_ref[...], preferred_element_type=jnp.float32)
         + jnp.dot(g2_ref[...], wb_ref[...],
                   preferred_element_type=jnp.float32)
         + b1_ref[...])
  hid = jnp.maximum(hid, 0.0)
  out = jnp.sum(hid * w2_ref[...], axis=1) + b2_ref[0, 0]
  out_ref[...] = out[None, :]


def _tc_mlp(g1, g2, waT, wbT, b1, w2, b2):
  grid = CP // MLP_BLK
  return pl.pallas_call(
      _mlp_body,
      grid=(grid,),
      in_specs=[
          pl.BlockSpec((MLP_BLK, 16), lambda i: (i, 0)),
          pl.BlockSpec((MLP_BLK, 16), lambda i: (i, 0)),
          pl.BlockSpec((16, 64), lambda i: (0, 0)),
          pl.BlockSpec((16, 64), lambda i: (0, 0)),
          pl.BlockSpec((1, 64), lambda i: (0, 0)),
          pl.BlockSpec((1, 64), lambda i: (0, 0)),
          pl.BlockSpec((1, 1), lambda i: (0, 0)),
      ],
      out_specs=pl.BlockSpec((1, MLP_BLK), lambda i: (i, 0)),
      out_shape=jax.ShapeDtypeStruct((grid, MLP_BLK), jnp.float32),
  )(g1, g2, waT, wbT, b1, w2, b2)


_agg_f2 = _make_agg(2)
_agg_f8 = _make_agg(8)
_cand_gather = _make_gather()


def kernel(x, mask, candidates, edges,
           W_rel1, b_rel1, W_root1, W_rel2, b_rel2, W_root2,
           W_u, b_u, W_f1, b_f1, W_f2, b_f2):
  e0 = edges[0].reshape(E // LANE, LANE)
  e1 = edges[1].reshape(E // LANE, LANE)
  zeros8 = jnp.zeros((N_CONN, 8), jnp.float32)

  xc = x[:N_CONN]
  xu_in = x[N_CONN:]

  agg1 = _agg_f2(x, e0, e1, zeros8[:, :2])
  h1 = _tc_layer(agg1, xc, W_rel1.T, b_rel1[None, :], W_root1.T, 2, 8)

  agg2 = _agg_f8(h1, e0, e1, zeros8)
  h2 = _tc_layer(agg2, h1, W_rel2.T, b_rel2[None, :], W_root2.T, 8, 16)

  xu = _tc_xu(xu_in, W_u.T, b_u[None, :])

  pad = CP - C
  c0 = jnp.concatenate([candidates[:, 0], jnp.zeros((pad,), jnp.int32)])
  c1 = jnp.concatenate([candidates[:, 1], jnp.zeros((pad,), jnp.int32)])
  g1, g2 = _cand_gather(h2, xu, c0, c1)

  outp = _tc_mlp(g1.reshape(CP, 16), g2.reshape(CP, 16),
                 W_f1[:, :16].T, W_f1[:, 16:].T, b_f1[None, :],
                 W_f2, b_f2[None, :])
  return outp.reshape(-1)[:C]


# speedup vs baseline: 25.2803x; 25.2803x over previous
import functools

import jax
import jax.numpy as jnp
from jax import lax
from jax.experimental import pallas as pl
from jax.experimental.pallas import tpu as pltpu
from jax.experimental.pallas import tpu_sc as plsc

N = 100000
N_CONN = 80000
N_UNC = N - N_CONN
E = 6400000
C = 1000000
F = 16

NC = 2
NS = 16
NW = NC * NS

LANE = 128
KSUB = 8
BLOCK = LANE * KSUB

_SC_PARAMS = pltpu.CompilerParams(use_tc_tiling_on_sc=False)


E_BLOCKS = E // BLOCK
E_PER, E_EXTRA = divmod(E_BLOCKS, NW)
ROWS_PER_TILE = N_CONN // NS


def _agg_body(h_hbm, e0_hbm, e1_hbm, zero_hbm, out_hbm,
              idx0_v, idx1_v, rows_v, agg_sh, gsem):
  cid = lax.axis_index("c")
  sid = lax.axis_index("s")
  wid = cid * NS + sid

  zslc = pl.ds(sid * ROWS_PER_TILE, ROWS_PER_TILE)
  pltpu.sync_copy(zero_hbm.at[zslc], agg_sh.at[zslc])
  plsc.subcore_barrier()

  start = wid * E_PER + jnp.minimum(wid, E_EXTRA)
  nblk = E_PER + jnp.where(wid < E_EXTRA, 1, 0)

  def blk(b, _):
    rowbase = (start + b) * KSUB
    pltpu.sync_copy(e0_hbm.at[pl.ds(rowbase, KSUB)], idx0_v)
    pltpu.sync_copy(e1_hbm.at[pl.ds(rowbase, KSUB)], idx1_v)
    for j in range(KSUB):
      pltpu.async_copy(h_hbm.at[idx0_v.at[j]], rows_v.at[j], gsem)
    for j in range(KSUB):
      pltpu.make_async_copy(h_hbm.at[idx0_v.at[j]], rows_v.at[j],
                            gsem).wait()
      pltpu.sync_copy(rows_v.at[j], agg_sh.at[idx1_v.at[j]], add=True)
    return 0

  lax.fori_loop(0, nblk, blk, 0)
  plsc.subcore_barrier()

  oslc = pl.ds(cid * N_CONN + sid * ROWS_PER_TILE, ROWS_PER_TILE)
  pltpu.sync_copy(agg_sh.at[zslc], out_hbm.at[oslc])


def _make_agg():
  mesh = plsc.VectorSubcoreMesh(core_axis_name="c", subcore_axis_name="s",
                                num_cores=NC, num_subcores=NS)
  return pl.kernel(
      _agg_body,
      out_type=jax.ShapeDtypeStruct((NC * N_CONN, F), jnp.float32),
      mesh=mesh,
      scratch_types=[
          pltpu.VMEM((KSUB, LANE), jnp.int32),
          pltpu.VMEM((KSUB, LANE), jnp.int32),
          pltpu.VMEM((KSUB, LANE, F), jnp.float32),
          pltpu.VMEM_SHARED((N_CONN, F), jnp.float32),
          pltpu.SemaphoreType.DMA,
      ],
      compiler_params=_SC_PARAMS,
  )


CP_BLOCKS = -(-C // BLOCK) + 1
CP = CP_BLOCKS * BLOCK
C_PER, C_EXTRA = divmod(CP_BLOCKS, NW)


def _gather_body(h2_hbm, xu_hbm, c0_hbm, c1_hbm, g1_hbm, g2_hbm,
                 idx0_v, idx1_v, rows1_v, rows2_v, gsem):
  cid = lax.axis_index("c")
  sid = lax.axis_index("s")
  wid = cid * NS + sid
  start = wid * C_PER + jnp.minimum(wid, C_EXTRA)
  nblk = C_PER + jnp.where(wid < C_EXTRA, 1, 0)

  def blk(b, _):
    base = (start + b) * BLOCK
    pltpu.sync_copy(c0_hbm.at[pl.ds(base, BLOCK)], idx0_v)
    pltpu.sync_copy(c1_hbm.at[pl.ds(base, BLOCK)], idx1_v)
    for j in range(KSUB):
      lslc = pl.ds(j * LANE, LANE)
      pltpu.async_copy(h2_hbm.at[idx0_v.at[lslc]], rows1_v.at[j], gsem)
      pltpu.async_copy(xu_hbm.at[idx1_v.at[lslc]], rows2_v.at[j], gsem)
    for j in range(KSUB):
      lslc = pl.ds(j * LANE, LANE)
      pltpu.make_async_copy(h2_hbm.at[idx0_v.at[lslc]], rows1_v.at[j],
                            gsem).wait()
      pltpu.make_async_copy(xu_hbm.at[idx1_v.at[lslc]], rows2_v.at[j],
                            gsem).wait()
    rowblk = (start + b) * KSUB
    pltpu.sync_copy(rows1_v, g1_hbm.at[pl.ds(rowblk, KSUB)])
    pltpu.sync_copy(rows2_v, g2_hbm.at[pl.ds(rowblk, KSUB)])
    return 0

  lax.fori_loop(0, nblk, blk, 0)


def _make_gather():
  mesh = plsc.VectorSubcoreMesh(core_axis_name="c", subcore_axis_name="s",
                                num_cores=NC, num_subcores=NS)
  return pl.kernel(
      _gather_body,
      out_type=(
          jax.ShapeDtypeStruct((CP_BLOCKS * KSUB, LANE, F), jnp.float32),
          jax.ShapeDtypeStruct((CP_BLOCKS * KSUB, LANE, F), jnp.float32),
      ),
      mesh=mesh,
      scratch_types=[
          pltpu.VMEM((BLOCK,), jnp.int32),
          pltpu.VMEM((BLOCK,), jnp.int32),
          pltpu.VMEM((KSUB, LANE, F), jnp.float32),
          pltpu.VMEM((KSUB, LANE, F), jnp.float32),
          pltpu.SemaphoreType.DMA,
      ],
      compiler_params=_SC_PARAMS,
  )


_make_agg = functools.lru_cache(None)(_make_agg)
_make_gather = functools.lru_cache(None)(_make_gather)


def _layer_body(agg0_ref, agg1_ref, h_ref, wr_ref, br_ref, wt_ref, out_ref):
  a = agg0_ref[...] + agg1_ref[...]
  z = (jnp.dot(a, wr_ref[...], preferred_element_type=jnp.float32)
       + jnp.dot(h_ref[...], wt_ref[...], preferred_element_type=jnp.float32)
       + br_ref[...])
  out_ref[...] = jnp.maximum(z, 0.0)


def _tc_layer(aggflat, h, wrT, br, wtT):
  m = h.shape[0]
  blk = 8000
  grid = m // blk
  nb = m // blk
  return pl.pallas_call(
      _layer_body,
      grid=(grid,),
      in_specs=[
          pl.BlockSpec((blk, F), lambda i: (i, 0)),
          pl.BlockSpec((blk, F), lambda i, _nb=nb: (i + _nb, 0)),
          pl.BlockSpec((blk, F), lambda i: (i, 0)),
          pl.BlockSpec((F, F), lambda i: (0, 0)),
          pl.BlockSpec((1, F), lambda i: (0, 0)),
          pl.BlockSpec((F, F), lambda i: (0, 0)),
      ],
      out_specs=pl.BlockSpec((blk, F), lambda i: (i, 0)),
      out_shape=jax.ShapeDtypeStruct((m, F), jnp.float32),
  )(aggflat, aggflat, h, wrT, br, wtT)


def _xu_body(x_ref, w_ref, b_ref, out_ref):
  out_ref[...] = (jnp.dot(x_ref[...], w_ref[...],
                          preferred_element_type=jnp.float32) + b_ref[...])


def _tc_xu(xu_in, wuT, bu):
  m = xu_in.shape[0]
  blk = 4000
  return pl.pallas_call(
      _xu_body,
      grid=(m // blk,),
      in_specs=[
          pl.BlockSpec((blk, F), lambda i: (i, 0)),
          pl.BlockSpec((F, F), lambda i: (0, 0)),
          pl.BlockSpec((1, F), lambda i: (0, 0)),
      ],
      out_specs=pl.BlockSpec((blk, F), lambda i: (i, 0)),
      out_shape=jax.ShapeDtypeStruct((m, F), jnp.float32),
  )(xu_in, wuT, bu)


MLP_BLK = 2048


def _mlp_body(g1_ref, g2_ref, wa_ref, wb_ref, b1_ref, w2_ref, b2_ref,
              out_ref):
  hid = (jnp.dot(g1_ref[...], wa_ref[...], preferred_element_type=jnp.float32)
         + jnp.dot(g2_ref[...], wb_ref[...],
                   preferred_element_type=jnp.float32)
         + b1_ref[...])
  hid = jnp.maximum(hid, 0.0)
  out = jnp.sum(hid * w2_ref[...], axis=1) + b2_ref[0, 0]
  out_ref[...] = out.reshape(MLP_BLK // 128, 128)


def _tc_mlp(g1, g2, waT, wbT, b1, w2, b2):
  grid = CP // MLP_BLK
  return pl.pallas_call(
      _mlp_body,
      grid=(grid,),
      in_specs=[
          pl.BlockSpec((MLP_BLK, F), lambda i: (i, 0)),
          pl.BlockSpec((MLP_BLK, F), lambda i: (i, 0)),
          pl.BlockSpec((F, 64), lambda i: (0, 0)),
          pl.BlockSpec((F, 64), lambda i: (0, 0)),
          pl.BlockSpec((1, 64), lambda i: (0, 0)),
          pl.BlockSpec((1, 64), lambda i: (0, 0)),
          pl.BlockSpec((1, 1), lambda i: (0, 0)),
      ],
      out_specs=pl.BlockSpec((MLP_BLK // 128, 128), lambda i: (i, 0)),
      out_shape=jax.ShapeDtypeStruct((CP // 128, 128), jnp.float32),
  )(g1, g2, waT, wbT, b1, w2, b2)


def _padw(wT, rows=F, cols=F):
  r, c = wT.shape
  return jnp.pad(wT, ((0, rows - r), (0, cols - c)))


def _padb(b, cols=F):
  return jnp.pad(b[None, :], ((0, 0), (0, cols - b.shape[0])))


def kernel(x, mask, candidates, edges,
           W_rel1, b_rel1, W_root1, W_rel2, b_rel2, W_root2,
           W_u, b_u, W_f1, b_f1, W_f2, b_f2):
  e0 = edges[0].reshape(E // LANE, LANE)
  e1 = edges[1].reshape(E // LANE, LANE)
  zeros16 = jnp.zeros((N_CONN, F), jnp.float32)

  x16 = jnp.pad(x, ((0, 0), (0, F - 2)))
  xc16 = x16[:N_CONN]
  xu_in16 = x16[N_CONN:]

  agg1 = _make_agg()(x16, e0, e1, zeros16)
  h1 = _tc_layer(agg1, xc16, _padw(W_rel1.T), _padb(b_rel1), _padw(W_root1.T))

  agg2 = _make_agg()(h1, e0, e1, zeros16)
  h2 = _tc_layer(agg2, h1, _padw(W_rel2.T), _padb(b_rel2), _padw(W_root2.T))

  xu = _tc_xu(xu_in16, _padw(W_u.T), _padb(b_u))

  pad = CP - C
  c0 = jnp.concatenate([candidates[:, 0], jnp.zeros((pad,), jnp.int32)])
  c1 = jnp.concatenate([candidates[:, 1], jnp.zeros((pad,), jnp.int32)])
  g1, g2 = _make_gather()(h2, xu, c0, c1)

  outp = _tc_mlp(g1.reshape(CP, F), g2.reshape(CP, F),
                 W_f1[:, :16].T, W_f1[:, 16:].T, b_f1[None, :],
                 W_f2, b_f2[None, :])
  return outp.reshape(-1)[:C]


# speedup vs baseline: 35.3689x; 1.3991x over previous
import functools

import jax
import jax.numpy as jnp
from jax import lax
from jax.experimental import pallas as pl
from jax.experimental.pallas import tpu as pltpu
from jax.experimental.pallas import tpu_sc as plsc

N = 100000
N_CONN = 80000
N_UNC = N - N_CONN
E = 6400000
C = 1000000
F = 16

NC = 2
NS = 16
NW = NC * NS

LANE = 128
KSUB = 8
BLOCK = LANE * KSUB

_SC_PARAMS = pltpu.CompilerParams(use_tc_tiling_on_sc=False)


E_BLOCKS = E // BLOCK
E_PER, E_EXTRA = divmod(E_BLOCKS, NW)
ROWS_PER_TILE = N_CONN // NS


def _agg_body(h_hbm, e0_hbm, e1_hbm, zero_hbm, out_hbm,
              idx0_v, idx1_v, rows_v, agg_sh, gsem):
  cid = lax.axis_index("c")
  sid = lax.axis_index("s")
  wid = cid * NS + sid

  zslc = pl.ds(sid * ROWS_PER_TILE, ROWS_PER_TILE)
  pltpu.sync_copy(zero_hbm.at[zslc], agg_sh.at[zslc])
  plsc.subcore_barrier()

  start = wid * E_PER + jnp.minimum(wid, E_EXTRA)
  nblk = E_PER + jnp.where(wid < E_EXTRA, 1, 0)

  def blk(b, _):
    rowbase = (start + b) * KSUB
    pltpu.sync_copy(e0_hbm.at[pl.ds(rowbase, KSUB)], idx0_v)
    pltpu.sync_copy(e1_hbm.at[pl.ds(rowbase, KSUB)], idx1_v)
    for j in range(KSUB):
      pltpu.async_copy(h_hbm.at[idx0_v.at[j]], rows_v.at[j], gsem)
    for j in range(KSUB):
      pltpu.make_async_copy(h_hbm.at[idx0_v.at[j]], rows_v.at[j],
                            gsem).wait()
      pltpu.sync_copy(rows_v.at[j], agg_sh.at[idx1_v.at[j]], add=True)
    return 0

  lax.fori_loop(0, nblk, blk, 0)
  plsc.subcore_barrier()

  oslc = pl.ds(cid * N_CONN + sid * ROWS_PER_TILE, ROWS_PER_TILE)
  pltpu.sync_copy(agg_sh.at[zslc], out_hbm.at[oslc])


def _make_agg():
  mesh = plsc.VectorSubcoreMesh(core_axis_name="c", subcore_axis_name="s",
                                num_cores=NC, num_subcores=NS)
  return pl.kernel(
      _agg_body,
      out_type=jax.ShapeDtypeStruct((NC * N_CONN, F), jnp.float32),
      mesh=mesh,
      scratch_types=[
          pltpu.VMEM((KSUB, LANE), jnp.int32),
          pltpu.VMEM((KSUB, LANE), jnp.int32),
          pltpu.VMEM((KSUB, LANE, F), jnp.float32),
          pltpu.VMEM_SHARED((N_CONN, F), jnp.float32),
          pltpu.SemaphoreType.DMA,
      ],
      compiler_params=_SC_PARAMS,
  )


CP_BLOCKS = -(-C // BLOCK) + 1
CP = CP_BLOCKS * BLOCK
C_PER, C_EXTRA = divmod(CP_BLOCKS, NW)


def _gather_body(h2_hbm, xu_hbm, c0_hbm, c1_hbm, g1_hbm, g2_hbm,
                 idx0_v, idx1_v, rows1_v, rows2_v, gsem):
  cid = lax.axis_index("c")
  sid = lax.axis_index("s")
  wid = cid * NS + sid
  start = wid * C_PER + jnp.minimum(wid, C_EXTRA)
  nblk = C_PER + jnp.where(wid < C_EXTRA, 1, 0)

  def blk(b, _):
    base = (start + b) * BLOCK
    pltpu.sync_copy(c0_hbm.at[pl.ds(base, BLOCK)], idx0_v)
    pltpu.sync_copy(c1_hbm.at[pl.ds(base, BLOCK)], idx1_v)
    for j in range(KSUB):
      lslc = pl.ds(j * LANE, LANE)
      pltpu.async_copy(h2_hbm.at[idx0_v.at[lslc]], rows1_v.at[j], gsem)
      pltpu.async_copy(xu_hbm.at[idx1_v.at[lslc]], rows2_v.at[j], gsem)
    for j in range(KSUB):
      lslc = pl.ds(j * LANE, LANE)
      pltpu.make_async_copy(h2_hbm.at[idx0_v.at[lslc]], rows1_v.at[j],
                            gsem).wait()
      pltpu.make_async_copy(xu_hbm.at[idx1_v.at[lslc]], rows2_v.at[j],
                            gsem).wait()
    rowblk = (start + b) * KSUB
    pltpu.sync_copy(rows1_v, g1_hbm.at[pl.ds(rowblk, KSUB)])
    pltpu.sync_copy(rows2_v, g2_hbm.at[pl.ds(rowblk, KSUB)])
    return 0

  lax.fori_loop(0, nblk, blk, 0)


def _make_gather():
  mesh = plsc.VectorSubcoreMesh(core_axis_name="c", subcore_axis_name="s",
                                num_cores=NC, num_subcores=NS)
  return pl.kernel(
      _gather_body,
      out_type=(
          jax.ShapeDtypeStruct((CP_BLOCKS * KSUB, LANE, F), jnp.float32),
          jax.ShapeDtypeStruct((CP_BLOCKS * KSUB, LANE, F), jnp.float32),
      ),
      mesh=mesh,
      scratch_types=[
          pltpu.VMEM((BLOCK,), jnp.int32),
          pltpu.VMEM((BLOCK,), jnp.int32),
          pltpu.VMEM((KSUB, LANE, F), jnp.float32),
          pltpu.VMEM((KSUB, LANE, F), jnp.float32),
          pltpu.SemaphoreType.DMA,
      ],
      compiler_params=_SC_PARAMS,
  )


_make_agg = functools.lru_cache(None)(_make_agg)
_make_gather = functools.lru_cache(None)(_make_gather)


PACK = 128 // F


def _layer_body(agg0_ref, agg1_ref, h_ref, wr_ref, br_ref, wt_ref, out_ref):
  a = agg0_ref[...] + agg1_ref[...]
  z = (jnp.dot(a, wr_ref[...], preferred_element_type=jnp.float32)
       + jnp.dot(h_ref[...], wt_ref[...], preferred_element_type=jnp.float32)
       + br_ref[...])
  out_ref[...] = jnp.maximum(z, 0.0)


def _tc_layer(aggflat, h, wrB, brB, wtB):
  m = h.shape[0]
  blk = 1000
  grid = m // blk
  nb = m // blk
  return pl.pallas_call(
      _layer_body,
      grid=(grid,),
      in_specs=[
          pl.BlockSpec((blk, 128), lambda i: (i, 0)),
          pl.BlockSpec((blk, 128), lambda i, _nb=nb: (i + _nb, 0)),
          pl.BlockSpec((blk, 128), lambda i: (i, 0)),
          pl.BlockSpec((128, 128), lambda i: (0, 0)),
          pl.BlockSpec((1, 128), lambda i: (0, 0)),
          pl.BlockSpec((128, 128), lambda i: (0, 0)),
      ],
      out_specs=pl.BlockSpec((blk, 128), lambda i: (i, 0)),
      out_shape=jax.ShapeDtypeStruct((m, 128), jnp.float32),
  )(aggflat, aggflat, h, wrB, brB, wtB)


def _xu_body(x_ref, w_ref, b_ref, out_ref):
  out_ref[...] = (jnp.dot(x_ref[...], w_ref[...],
                          preferred_element_type=jnp.float32) + b_ref[...])


def _tc_xu(xu_in, wuB, buB):
  m = xu_in.shape[0]
  blk = m
  return pl.pallas_call(
      _xu_body,
      grid=(m // blk,),
      in_specs=[
          pl.BlockSpec((blk, 128), lambda i: (i, 0)),
          pl.BlockSpec((128, 128), lambda i: (0, 0)),
          pl.BlockSpec((1, 128), lambda i: (0, 0)),
      ],
      out_specs=pl.BlockSpec((blk, 128), lambda i: (i, 0)),
      out_shape=jax.ShapeDtypeStruct((m, 128), jnp.float32),
  )(xu_in, wuB, buB)


MLP_BLK = 256


def _mlp_body(g1_ref, g2_ref, wa_ref, wb_ref, b1_ref, w2_ref, b2_ref,
              out_ref):
  hid = (jnp.dot(g1_ref[...], wa_ref[...], preferred_element_type=jnp.float32)
         + jnp.dot(g2_ref[...], wb_ref[...],
                   preferred_element_type=jnp.float32)
         + b1_ref[...])
  hid = jnp.maximum(hid, 0.0)
  out_ref[...] = (jnp.dot(hid, w2_ref[...], preferred_element_type=jnp.float32)
                  + b2_ref[...])


def _tc_mlp(g1, g2, waB, wbB, b1B, w2B, b2):
  mp = CP // PACK
  grid = mp // MLP_BLK
  return pl.pallas_call(
      _mlp_body,
      grid=(grid,),
      in_specs=[
          pl.BlockSpec((MLP_BLK, 128), lambda i: (i, 0)),
          pl.BlockSpec((MLP_BLK, 128), lambda i: (i, 0)),
          pl.BlockSpec((128, PACK * 64), lambda i: (0, 0)),
          pl.BlockSpec((128, PACK * 64), lambda i: (0, 0)),
          pl.BlockSpec((1, PACK * 64), lambda i: (0, 0)),
          pl.BlockSpec((PACK * 64, PACK), lambda i: (0, 0)),
          pl.BlockSpec((1, PACK), lambda i: (0, 0)),
      ],
      out_specs=pl.BlockSpec((MLP_BLK, PACK), lambda i: (i, 0)),
      out_shape=jax.ShapeDtypeStruct((mp, PACK), jnp.float32),
  )(g1, g2, waB, wbB, b1B, w2B, b2)


def _padw(wT, rows=F, cols=F):
  r, c = wT.shape
  return jnp.pad(wT, ((0, rows - r), (0, cols - c)))


def _bdiag(wT):
  return jnp.kron(jnp.eye(PACK, dtype=jnp.float32), wT)


def _bdb(b, width=F):
  bp = jnp.pad(b, (0, width - b.shape[0]))
  return jnp.tile(bp, PACK)[None, :]


def kernel(x, mask, candidates, edges,
           W_rel1, b_rel1, W_root1, W_rel2, b_rel2, W_root2,
           W_u, b_u, W_f1, b_f1, W_f2, b_f2):
  e0 = edges[0].reshape(E // LANE, LANE)
  e1 = edges[1].reshape(E // LANE, LANE)
  zeros16 = jnp.zeros((N_CONN, F), jnp.float32)

  x16 = jnp.pad(x, ((0, 0), (0, F - 2)))
  xc_p = x16[:N_CONN].reshape(N_CONN // PACK, 128)
  xu_in_p = x16[N_CONN:].reshape(N_UNC // PACK, 128)

  agg1 = _make_agg()(x16, e0, e1, zeros16)
  h1_p = _tc_layer(agg1.reshape(NC * N_CONN // PACK, 128), xc_p,
                   _bdiag(_padw(W_rel1.T)), _bdb(b_rel1),
                   _bdiag(_padw(W_root1.T)))

  agg2 = _make_agg()(h1_p.reshape(N_CONN, F), e0, e1, zeros16)
  h2_p = _tc_layer(agg2.reshape(NC * N_CONN // PACK, 128), h1_p,
                   _bdiag(_padw(W_rel2.T)), _bdb(b_rel2),
                   _bdiag(_padw(W_root2.T)))

  xu_p = _tc_xu(xu_in_p, _bdiag(_padw(W_u.T)), _bdb(b_u))

  pad = CP - C
  c0 = jnp.concatenate([candidates[:, 0], jnp.zeros((pad,), jnp.int32)])
  c1 = jnp.concatenate([candidates[:, 1], jnp.zeros((pad,), jnp.int32)])
  g1, g2 = _make_gather()(h2_p.reshape(N_CONN, F), xu_p.reshape(N_UNC, F),
                          c0, c1)

  outp = _tc_mlp(g1.reshape(CP // PACK, 128), g2.reshape(CP // PACK, 128),
                 _bdiag(W_f1[:, :16].T), _bdiag(W_f1[:, 16:].T),
                 _bdb(b_f1, 64),
                 _bdiag(W_f2.T), jnp.tile(b_f2, PACK)[None, :])
  return outp.reshape(-1)[:C]


# speedup vs baseline: 45.4487x; 1.2850x over previous
import functools

import jax
import jax.numpy as jnp
from jax import lax
from jax.experimental import pallas as pl
from jax.experimental.pallas import tpu as pltpu
from jax.experimental.pallas import tpu_sc as plsc

N = 100000
N_CONN = 80000
N_UNC = N - N_CONN
E = 6400000
C = 1000000
F = 16

NC = 2
NS = 16
NW = NC * NS

LANE = 128
KSUB = 8
BLOCK = LANE * KSUB

_SC_PARAMS = pltpu.CompilerParams(use_tc_tiling_on_sc=False)


KA = 16
E_BLOCKS = E // (LANE * KA)
E_PER, E_EXTRA = divmod(E_BLOCKS, NW)
ROWS_PER_TILE = N_CONN // NS


def _agg_body(h_hbm, e0_hbm, e1_hbm, zero_hbm, out_hbm,
              idx0_v, idx1_v, rows_v, agg_sh, gsem, ssem):
  cid = lax.axis_index("c")
  sid = lax.axis_index("s")
  wid = cid * NS + sid

  zslc = pl.ds(sid * ROWS_PER_TILE, ROWS_PER_TILE)
  pltpu.sync_copy(zero_hbm.at[zslc], agg_sh.at[zslc])
  plsc.subcore_barrier()

  start = wid * E_PER + jnp.minimum(wid, E_EXTRA)
  nblk = E_PER + jnp.where(wid < E_EXTRA, 1, 0)

  def blk(b, _):
    rowbase = (start + b) * KA
    pltpu.sync_copy(e0_hbm.at[pl.ds(rowbase, KA)], idx0_v)
    pltpu.sync_copy(e1_hbm.at[pl.ds(rowbase, KA)], idx1_v)
    for j in range(KA):
      pltpu.async_copy(h_hbm.at[idx0_v.at[j]], rows_v.at[j], gsem)
    for j in range(KA):
      pltpu.make_async_copy(h_hbm.at[idx0_v.at[j]], rows_v.at[j],
                            gsem).wait()
      pltpu.async_copy(rows_v.at[j], agg_sh.at[idx1_v.at[j]], ssem,
                       add=True)
    for j in range(KA):
      pltpu.make_async_copy(rows_v.at[j], agg_sh.at[idx1_v.at[j]],
                            ssem).wait()
    return 0

  lax.fori_loop(0, nblk, blk, 0)
  plsc.subcore_barrier()

  oslc = pl.ds(cid * N_CONN + sid * ROWS_PER_TILE, ROWS_PER_TILE)
  pltpu.sync_copy(agg_sh.at[zslc], out_hbm.at[oslc])


def _make_agg():
  mesh = plsc.VectorSubcoreMesh(core_axis_name="c", subcore_axis_name="s",
                                num_cores=NC, num_subcores=NS)
  return pl.kernel(
      _agg_body,
      out_type=jax.ShapeDtypeStruct((NC * N_CONN, F), jnp.float32),
      mesh=mesh,
      scratch_types=[
          pltpu.VMEM((KA, LANE), jnp.int32),
          pltpu.VMEM((KA, LANE), jnp.int32),
          pltpu.VMEM((KA, LANE, F), jnp.float32),
          pltpu.VMEM_SHARED((N_CONN, F), jnp.float32),
          pltpu.SemaphoreType.DMA,
          pltpu.SemaphoreType.DMA,
      ],
      compiler_params=_SC_PARAMS,
  )


CP_BLOCKS = -(-C // BLOCK) + 1
CP = CP_BLOCKS * BLOCK
C_PER, C_EXTRA = divmod(CP_BLOCKS, NW)


def _gather_body(h2_hbm, xu_hbm, c0_hbm, c1_hbm, g1_hbm, g2_hbm,
                 idx0_v, idx1_v, rows1_v, rows2_v, gsem):
  cid = lax.axis_index("c")
  sid = lax.axis_index("s")
  wid = cid * NS + sid
  start = wid * C_PER + jnp.minimum(wid, C_EXTRA)
  nblk = C_PER + jnp.where(wid < C_EXTRA, 1, 0)

  def blk(b, _):
    base = (start + b) * BLOCK
    pltpu.sync_copy(c0_hbm.at[pl.ds(base, BLOCK)], idx0_v)
    pltpu.sync_copy(c1_hbm.at[pl.ds(base, BLOCK)], idx1_v)
    for j in range(KSUB):
      lslc = pl.ds(j * LANE, LANE)
      pltpu.async_copy(h2_hbm.at[idx0_v.at[lslc]], rows1_v.at[j], gsem)
      pltpu.async_copy(xu_hbm.at[idx1_v.at[lslc]], rows2_v.at[j], gsem)
    for j in range(KSUB):
      lslc = pl.ds(j * LANE, LANE)
      pltpu.make_async_copy(h2_hbm.at[idx0_v.at[lslc]], rows1_v.at[j],
                            gsem).wait()
      pltpu.make_async_copy(xu_hbm.at[idx1_v.at[lslc]], rows2_v.at[j],
                            gsem).wait()
    rowblk = (start + b) * KSUB
    pltpu.sync_copy(rows1_v, g1_hbm.at[pl.ds(rowblk, KSUB)])
    pltpu.sync_copy(rows2_v, g2_hbm.at[pl.ds(rowblk, KSUB)])
    return 0

  lax.fori_loop(0, nblk, blk, 0)


def _make_gather():
  mesh = plsc.VectorSubcoreMesh(core_axis_name="c", subcore_axis_name="s",
                                num_cores=NC, num_subcores=NS)
  return pl.kernel(
      _gather_body,
      out_type=(
          jax.ShapeDtypeStruct((CP_BLOCKS * KSUB, LANE, F), jnp.float32),
          jax.ShapeDtypeStruct((CP_BLOCKS * KSUB, LANE, F), jnp.float32),
      ),
      mesh=mesh,
      scratch_types=[
          pltpu.VMEM((BLOCK,), jnp.int32),
          pltpu.VMEM((BLOCK,), jnp.int32),
          pltpu.VMEM((KSUB, LANE, F), jnp.float32),
          pltpu.VMEM((KSUB, LANE, F), jnp.float32),
          pltpu.SemaphoreType.DMA,
      ],
      compiler_params=_SC_PARAMS,
  )


_make_agg = functools.lru_cache(None)(_make_agg)
_make_gather = functools.lru_cache(None)(_make_gather)


PACK = 128 // F


def _layer_body(agg0_ref, agg1_ref, h_ref, wr_ref, br_ref, wt_ref, out_ref):
  a = agg0_ref[...] + agg1_ref[...]
  z = (jnp.dot(a, wr_ref[...], preferred_element_type=jnp.float32)
       + jnp.dot(h_ref[...], wt_ref[...], preferred_element_type=jnp.float32)
       + br_ref[...])
  out_ref[...] = jnp.maximum(z, 0.0)


def _tc_layer(aggflat, h, wrB, brB, wtB):
  m = h.shape[0]
  blk = 1000
  grid = m // blk
  nb = m // blk
  return pl.pallas_call(
      _layer_body,
      grid=(grid,),
      in_specs=[
          pl.BlockSpec((blk, 128), lambda i: (i, 0)),
          pl.BlockSpec((blk, 128), lambda i, _nb=nb: (i + _nb, 0)),
          pl.BlockSpec((blk, 128), lambda i: (i, 0)),
          pl.BlockSpec((128, 128), lambda i: (0, 0)),
          pl.BlockSpec((1, 128), lambda i: (0, 0)),
          pl.BlockSpec((128, 128), lambda i: (0, 0)),
      ],
      out_specs=pl.BlockSpec((blk, 128), lambda i: (i, 0)),
      out_shape=jax.ShapeDtypeStruct((m, 128), jnp.float32),
  )(aggflat, aggflat, h, wrB, brB, wtB)


def _xu_body(x_ref, w_ref, b_ref, out_ref):
  out_ref[...] = (jnp.dot(x_ref[...], w_ref[...],
                          preferred_element_type=jnp.float32) + b_ref[...])


def _tc_xu(xu_in, wuB, buB):
  m = xu_in.shape[0]
  blk = m
  return pl.pallas_call(
      _xu_body,
      grid=(m // blk,),
      in_specs=[
          pl.BlockSpec((blk, 128), lambda i: (i, 0)),
          pl.BlockSpec((128, 128), lambda i: (0, 0)),
          pl.BlockSpec((1, 128), lambda i: (0, 0)),
      ],
      out_specs=pl.BlockSpec((blk, 128), lambda i: (i, 0)),
      out_shape=jax.ShapeDtypeStruct((m, 128), jnp.float32),
  )(xu_in, wuB, buB)


MLP_BLK = 256


def _mlp_body(g1_ref, g2_ref, wa_ref, wb_ref, b1_ref, w2_ref, b2_ref,
              out_ref):
  hid = (jnp.dot(g1_ref[...], wa_ref[...], preferred_element_type=jnp.float32)
         + jnp.dot(g2_ref[...], wb_ref[...],
                   preferred_element_type=jnp.float32)
         + b1_ref[...])
  hid = jnp.maximum(hid, 0.0)
  out_ref[...] = (jnp.dot(hid, w2_ref[...], preferred_element_type=jnp.float32)
                  + b2_ref[...])


def _tc_mlp(g1, g2, waB, wbB, b1B, w2B, b2):
  mp = CP // PACK
  grid = mp // MLP_BLK
  return pl.pallas_call(
      _mlp_body,
      grid=(grid,),
      in_specs=[
          pl.BlockSpec((MLP_BLK, 128), lambda i: (i, 0)),
          pl.BlockSpec((MLP_BLK, 128), lambda i: (i, 0)),
          pl.BlockSpec((128, PACK * 64), lambda i: (0, 0)),
          pl.BlockSpec((128, PACK * 64), lambda i: (0, 0)),
          pl.BlockSpec((1, PACK * 64), lambda i: (0, 0)),
          pl.BlockSpec((PACK * 64, PACK), lambda i: (0, 0)),
          pl.BlockSpec((1, PACK), lambda i: (0, 0)),
      ],
      out_specs=pl.BlockSpec((MLP_BLK, PACK), lambda i: (i, 0)),
      out_shape=jax.ShapeDtypeStruct((mp, PACK), jnp.float32),
  )(g1, g2, waB, wbB, b1B, w2B, b2)


def _padw(wT, rows=F, cols=F):
  r, c = wT.shape
  return jnp.pad(wT, ((0, rows - r), (0, cols - c)))


def _bdiag(wT):
  return jnp.kron(jnp.eye(PACK, dtype=jnp.float32), wT)


def _bdb(b, width=F):
  bp = jnp.pad(b, (0, width - b.shape[0]))
  return jnp.tile(bp, PACK)[None, :]


def kernel(x, mask, candidates, edges,
           W_rel1, b_rel1, W_root1, W_rel2, b_rel2, W_root2,
           W_u, b_u, W_f1, b_f1, W_f2, b_f2):
  e0 = edges[0].reshape(E // LANE, LANE)
  e1 = edges[1].reshape(E // LANE, LANE)
  zeros16 = jnp.zeros((N_CONN, F), jnp.float32)

  x16 = jnp.pad(x, ((0, 0), (0, F - 2)))
  xc_p = x16[:N_CONN].reshape(N_CONN // PACK, 128)
  xu_in_p = x16[N_CONN:].reshape(N_UNC // PACK, 128)

  agg1 = _make_agg()(x16, e0, e1, zeros16)
  h1_p = _tc_layer(agg1.reshape(NC * N_CONN // PACK, 128), xc_p,
                   _bdiag(_padw(W_rel1.T)), _bdb(b_rel1),
                   _bdiag(_padw(W_root1.T)))

  agg2 = _make_agg()(h1_p.reshape(N_CONN, F), e0, e1, zeros16)
  h2_p = _tc_layer(agg2.reshape(NC * N_CONN // PACK, 128), h1_p,
                   _bdiag(_padw(W_rel2.T)), _bdb(b_rel2),
                   _bdiag(_padw(W_root2.T)))

  xu_p = _tc_xu(xu_in_p, _bdiag(_padw(W_u.T)), _bdb(b_u))

  pad = CP - C
  c0 = jnp.concatenate([candidates[:, 0], jnp.zeros((pad,), jnp.int32)])
  c1 = jnp.concatenate([candidates[:, 1], jnp.zeros((pad,), jnp.int32)])
  g1, g2 = _make_gather()(h2_p.reshape(N_CONN, F), xu_p.reshape(N_UNC, F),
                          c0, c1)

  outp = _tc_mlp(g1.reshape(CP // PACK, 128), g2.reshape(CP // PACK, 128),
                 _bdiag(W_f1[:, :16].T), _bdiag(W_f1[:, 16:].T),
                 _bdb(b_f1, 64),
                 _bdiag(W_f2.T), jnp.tile(b_f2, PACK)[None, :])
  return outp.reshape(-1)[:C]


# speedup vs baseline: 49.0721x; 1.0797x over previous
import functools

import jax
import jax.numpy as jnp
from jax import lax
from jax.experimental import pallas as pl
from jax.experimental.pallas import tpu as pltpu
from jax.experimental.pallas import tpu_sc as plsc

N = 100000
N_CONN = 80000
N_UNC = N - N_CONN
E = 6400000
C = 1000000
F = 16

NC = 2
NS = 16
NW = NC * NS

LANE = 128
KSUB = 8
BLOCK = LANE * KSUB

_SC_PARAMS = pltpu.CompilerParams(use_tc_tiling_on_sc=False)


KA = 8
E_BLOCKS = E // (LANE * KA)
E_PER, E_EXTRA = divmod(E_BLOCKS, NW)
ROWS_PER_TILE = N_CONN // NS


NBUF = 2
NSEC = 2 * (-(-(E_PER + 1) // 2))


def _agg_body(h_hbm, e0_hbm, e1_hbm, zero_hbm, out_hbm,
              i0_0, i1_0, r_0, i0_1, i1_1, r_1, agg_sh,
              gs_0, gs_1, ss_0, ss_1):
  cid = lax.axis_index("c")
  sid = lax.axis_index("s")
  wid = cid * NS + sid

  bufs = [(i0_0, i1_0, r_0, gs_0, ss_0),
          (i0_1, i1_1, r_1, gs_1, ss_1)]

  zslc = pl.ds(sid * ROWS_PER_TILE, ROWS_PER_TILE)
  pltpu.sync_copy(zero_hbm.at[zslc], agg_sh.at[zslc])
  plsc.subcore_barrier()

  start = wid * E_PER + jnp.minimum(wid, E_EXTRA)
  nblk = E_PER + jnp.where(wid < E_EXTRA, 1, 0)

  def load_and_fire(buf, b):
    i0, i1, rows, gs, _ = buf
    rowbase = (start + b) * KA
    pltpu.sync_copy(e0_hbm.at[pl.ds(rowbase, KA)], i0)
    pltpu.sync_copy(e1_hbm.at[pl.ds(rowbase, KA)], i1)
    for j in range(KA):
      pltpu.async_copy(h_hbm.at[i0.at[j]], rows.at[j], gs)

  def wait_fire(buf):
    i0, i1, rows, gs, ss = buf
    for j in range(KA):
      pltpu.make_async_copy(h_hbm.at[i0.at[j]], rows.at[j], gs).wait()
      pltpu.async_copy(rows.at[j], agg_sh.at[i1.at[j]], ss, add=True)

  def drain(buf):
    _, i1, rows, _, ss = buf
    for j in range(KA):
      pltpu.make_async_copy(rows.at[j], agg_sh.at[i1.at[j]], ss).wait()

  load_and_fire(bufs[0], 0)

  def body(t, _):
    for k in (1, 2):
      n = 2 * t + k
      buf = bufs[k % 2]
      bufw = bufs[(k - 1) % 2]
      if k == 2:
        cond_d = (2 * t) < nblk
      else:
        cond_d = jnp.logical_and(t > 0, (2 * t + k - 2) < nblk)

      @pl.when(cond_d)
      def _():
        drain(buf)

      @pl.when(n < nblk)
      def _():
        load_and_fire(buf, n)

      @pl.when(n - 1 < nblk)
      def _():
        wait_fire(bufw)

    return 0

  lax.fori_loop(0, NSEC // 2, body, 0)
  @pl.when(wid < E_EXTRA)
  def _():
    drain(bufs[1])
  plsc.subcore_barrier()

  oslc = pl.ds(cid * N_CONN + sid * ROWS_PER_TILE, ROWS_PER_TILE)
  pltpu.sync_copy(agg_sh.at[zslc], out_hbm.at[oslc])


def _make_agg():
  mesh = plsc.VectorSubcoreMesh(core_axis_name="c", subcore_axis_name="s",
                                num_cores=NC, num_subcores=NS)
  return pl.kernel(
      _agg_body,
      out_type=jax.ShapeDtypeStruct((NC * N_CONN, F), jnp.float32),
      mesh=mesh,
      scratch_types=(
          [pltpu.VMEM((KA, LANE), jnp.int32),
           pltpu.VMEM((KA, LANE), jnp.int32),
           pltpu.VMEM((KA, LANE, F), jnp.float32)] * NBUF
          + [pltpu.VMEM_SHARED((N_CONN, F), jnp.float32)]
          + [pltpu.SemaphoreType.DMA] * (2 * NBUF)
      ),
      compiler_params=_SC_PARAMS,
  )


CP_BLOCKS = -(-C // BLOCK) + 1
CP = CP_BLOCKS * BLOCK
C_PER, C_EXTRA = divmod(CP_BLOCKS, NW)


def _gather_body(h2_hbm, xu_hbm, c0_hbm, c1_hbm, g1_hbm, g2_hbm,
                 idx0_v, idx1_v, rows1_v, rows2_v, gsem):
  cid = lax.axis_index("c")
  sid = lax.axis_index("s")
  wid = cid * NS + sid
  start = wid * C_PER + jnp.minimum(wid, C_EXTRA)
  nblk = C_PER + jnp.where(wid < C_EXTRA, 1, 0)

  def blk(b, _):
    base = (start + b) * BLOCK
    pltpu.sync_copy(c0_hbm.at[pl.ds(base, BLOCK)], idx0_v)
    pltpu.sync_copy(c1_hbm.at[pl.ds(base, BLOCK)], idx1_v)
    for j in range(KSUB):
      lslc = pl.ds(j * LANE, LANE)
      pltpu.async_copy(h2_hbm.at[idx0_v.at[lslc]], rows1_v.at[j], gsem)
      pltpu.async_copy(xu_hbm.at[idx1_v.at[lslc]], rows2_v.at[j], gsem)
    for j in range(KSUB):
      lslc = pl.ds(j * LANE, LANE)
      pltpu.make_async_copy(h2_hbm.at[idx0_v.at[lslc]], rows1_v.at[j],
                            gsem).wait()
      pltpu.make_async_copy(xu_hbm.at[idx1_v.at[lslc]], rows2_v.at[j],
                            gsem).wait()
    rowblk = (start + b) * KSUB
    pltpu.sync_copy(rows1_v, g1_hbm.at[pl.ds(rowblk, KSUB)])
    pltpu.sync_copy(rows2_v, g2_hbm.at[pl.ds(rowblk, KSUB)])
    return 0

  lax.fori_loop(0, nblk, blk, 0)


def _make_gather():
  mesh = plsc.VectorSubcoreMesh(core_axis_name="c", subcore_axis_name="s",
                                num_cores=NC, num_subcores=NS)
  return pl.kernel(
      _gather_body,
      out_type=(
          jax.ShapeDtypeStruct((CP_BLOCKS * KSUB, LANE, F), jnp.float32),
          jax.ShapeDtypeStruct((CP_BLOCKS * KSUB, LANE, F), jnp.float32),
      ),
      mesh=mesh,
      scratch_types=[
          pltpu.VMEM((BLOCK,), jnp.int32),
          pltpu.VMEM((BLOCK,), jnp.int32),
          pltpu.VMEM((KSUB, LANE, F), jnp.float32),
          pltpu.VMEM((KSUB, LANE, F), jnp.float32),
          pltpu.SemaphoreType.DMA,
      ],
      compiler_params=_SC_PARAMS,
  )


_make_agg = functools.lru_cache(None)(_make_agg)
_make_gather = functools.lru_cache(None)(_make_gather)


PACK = 128 // F


def _layer_body(agg0_ref, agg1_ref, h_ref, wr_ref, br_ref, wt_ref, out_ref):
  a = agg0_ref[...] + agg1_ref[...]
  z = (jnp.dot(a, wr_ref[...], preferred_element_type=jnp.float32)
       + jnp.dot(h_ref[...], wt_ref[...], preferred_element_type=jnp.float32)
       + br_ref[...])
  out_ref[...] = jnp.maximum(z, 0.0)


def _tc_layer(aggflat, h, wrB, brB, wtB):
  m = h.shape[0]
  blk = 1000
  grid = m // blk
  nb = m // blk
  return pl.pallas_call(
      _layer_body,
      grid=(grid,),
      in_specs=[
          pl.BlockSpec((blk, 128), lambda i: (i, 0)),
          pl.BlockSpec((blk, 128), lambda i, _nb=nb: (i + _nb, 0)),
          pl.BlockSpec((blk, 128), lambda i: (i, 0)),
          pl.BlockSpec((128, 128), lambda i: (0, 0)),
          pl.BlockSpec((1, 128), lambda i: (0, 0)),
          pl.BlockSpec((128, 128), lambda i: (0, 0)),
      ],
      out_specs=pl.BlockSpec((blk, 128), lambda i: (i, 0)),
      out_shape=jax.ShapeDtypeStruct((m, 128), jnp.float32),
  )(aggflat, aggflat, h, wrB, brB, wtB)


def _xu_body(x_ref, w_ref, b_ref, out_ref):
  out_ref[...] = (jnp.dot(x_ref[...], w_ref[...],
                          preferred_element_type=jnp.float32) + b_ref[...])


def _tc_xu(xu_in, wuB, buB):
  m = xu_in.shape[0]
  blk = m
  return pl.pallas_call(
      _xu_body,
      grid=(m // blk,),
      in_specs=[
          pl.BlockSpec((blk, 128), lambda i: (i, 0)),
          pl.BlockSpec((128, 128), lambda i: (0, 0)),
          pl.BlockSpec((1, 128), lambda i: (0, 0)),
      ],
      out_specs=pl.BlockSpec((blk, 128), lambda i: (i, 0)),
      out_shape=jax.ShapeDtypeStruct((m, 128), jnp.float32),
  )(xu_in, wuB, buB)


MLP_BLK = 256


def _mlp_body(g1_ref, g2_ref, wa_ref, wb_ref, b1_ref, w2_ref, b2_ref,
              out_ref):
  hid = (jnp.dot(g1_ref[...], wa_ref[...], preferred_element_type=jnp.float32)
         + jnp.dot(g2_ref[...], wb_ref[...],
                   preferred_element_type=jnp.float32)
         + b1_ref[...])
  hid = jnp.maximum(hid, 0.0)
  out_ref[...] = (jnp.dot(hid, w2_ref[...], preferred_element_type=jnp.float32)
                  + b2_ref[...])


def _tc_mlp(g1, g2, waB, wbB, b1B, w2B, b2):
  mp = CP // PACK
  grid = mp // MLP_BLK
  return pl.pallas_call(
      _mlp_body,
      grid=(grid,),
      in_specs=[
          pl.BlockSpec((MLP_BLK, 128), lambda i: (i, 0)),
          pl.BlockSpec((MLP_BLK, 128), lambda i: (i, 0)),
          pl.BlockSpec((128, PACK * 64), lambda i: (0, 0)),
          pl.BlockSpec((128, PACK * 64), lambda i: (0, 0)),
          pl.BlockSpec((1, PACK * 64), lambda i: (0, 0)),
          pl.BlockSpec((PACK * 64, PACK), lambda i: (0, 0)),
          pl.BlockSpec((1, PACK), lambda i: (0, 0)),
      ],
      out_specs=pl.BlockSpec((MLP_BLK, PACK), lambda i: (i, 0)),
      out_shape=jax.ShapeDtypeStruct((mp, PACK), jnp.float32),
  )(g1, g2, waB, wbB, b1B, w2B, b2)


def _padw(wT, rows=F, cols=F):
  r, c = wT.shape
  return jnp.pad(wT, ((0, rows - r), (0, cols - c)))


def _bdiag(wT):
  return jnp.kron(jnp.eye(PACK, dtype=jnp.float32), wT)


def _bdb(b, width=F):
  bp = jnp.pad(b, (0, width - b.shape[0]))
  return jnp.tile(bp, PACK)[None, :]


def kernel(x, mask, candidates, edges,
           W_rel1, b_rel1, W_root1, W_rel2, b_rel2, W_root2,
           W_u, b_u, W_f1, b_f1, W_f2, b_f2):
  e0 = edges[0].reshape(E // LANE, LANE)
  e1 = edges[1].reshape(E // LANE, LANE)
  zeros16 = jnp.zeros((N_CONN, F), jnp.float32)

  x16 = jnp.pad(x, ((0, 0), (0, F - 2)))
  xc_p = x16[:N_CONN].reshape(N_CONN // PACK, 128)
  xu_in_p = x16[N_CONN:].reshape(N_UNC // PACK, 128)

  agg1 = _make_agg()(x16, e0, e1, zeros16)
  h1_p = _tc_layer(agg1.reshape(NC * N_CONN // PACK, 128), xc_p,
                   _bdiag(_padw(W_rel1.T)), _bdb(b_rel1),
                   _bdiag(_padw(W_root1.T)))

  agg2 = _make_agg()(h1_p.reshape(N_CONN, F), e0, e1, zeros16)
  h2_p = _tc_layer(agg2.reshape(NC * N_CONN // PACK, 128), h1_p,
                   _bdiag(_padw(W_rel2.T)), _bdb(b_rel2),
                   _bdiag(_padw(W_root2.T)))

  xu_p = _tc_xu(xu_in_p, _bdiag(_padw(W_u.T)), _bdb(b_u))

  pad = CP - C
  c0 = jnp.concatenate([candidates[:, 0], jnp.zeros((pad,), jnp.int32)])
  c1 = jnp.concatenate([candidates[:, 1], jnp.zeros((pad,), jnp.int32)])
  g1, g2 = _make_gather()(h2_p.reshape(N_CONN, F), xu_p.reshape(N_UNC, F),
                          c0, c1)

  outp = _tc_mlp(g1.reshape(CP // PACK, 128), g2.reshape(CP // PACK, 128),
                 _bdiag(W_f1[:, :16].T), _bdiag(W_f1[:, 16:].T),
                 _bdb(b_f1, 64),
                 _bdiag(W_f2.T), jnp.tile(b_f2, PACK)[None, :])
  return outp.reshape(-1)[:C]


# speedup vs baseline: 57.6895x; 1.1756x over previous
import functools

import jax
import jax.numpy as jnp
from jax import lax
from jax.experimental import pallas as pl
from jax.experimental.pallas import tpu as pltpu
from jax.experimental.pallas import tpu_sc as plsc

N = 100000
N_CONN = 80000
N_UNC = N - N_CONN
E = 6400000
C = 1000000
F = 16

NC = 2
NS = 16
NW = NC * NS

LANE = 128
KSUB = 8
BLOCK = LANE * KSUB

_SC_PARAMS = pltpu.CompilerParams(use_tc_tiling_on_sc=False)


KA = 8
E_BLOCKS = E // (LANE * KA)
E_PER, E_EXTRA = divmod(E_BLOCKS, NW)
ROWS_PER_TILE = N_CONN // NS


NSEC = 4 * (-(-(E_PER + 1) // 4))


def _agg_body(h_hbm, e0_hbm, e1_hbm, zero_hbm, out_hbm,
              r_0, r_1, i0_0, i1_0, i0_1, i1_1, i0_2, i1_2, i0_3, i1_3,
              agg_sh, gs_0, gs_1, ss_0, ss_1, is_0, is_1, is_2, is_3):
  cid = lax.axis_index("c")
  sid = lax.axis_index("s")
  wid = cid * NS + sid

  rows = [r_0, r_1]
  gsem = [gs_0, gs_1]
  ssem = [ss_0, ss_1]
  idx = [(i0_0, i1_0), (i0_1, i1_1), (i0_2, i1_2), (i0_3, i1_3)]
  isem = [is_0, is_1, is_2, is_3]

  zslc = pl.ds(sid * ROWS_PER_TILE, ROWS_PER_TILE)
  pltpu.sync_copy(zero_hbm.at[zslc], agg_sh.at[zslc])
  plsc.subcore_barrier()

  start = wid * E_PER + jnp.minimum(wid, E_EXTRA)
  nblk = E_PER + jnp.where(wid < E_EXTRA, 1, 0)

  def idx_start(s, b):
    i0, i1 = idx[s]
    rowbase = (start + b) * KA
    pltpu.async_copy(e0_hbm.at[pl.ds(rowbase, KA)], i0, isem[s])
    pltpu.async_copy(e1_hbm.at[pl.ds(rowbase, KA)], i1, isem[s])

  def idx_wait(s, b):
    i0, i1 = idx[s]
    rowbase = (start + b) * KA
    pltpu.make_async_copy(e0_hbm.at[pl.ds(rowbase, KA)], i0, isem[s]).wait()
    pltpu.make_async_copy(e1_hbm.at[pl.ds(rowbase, KA)], i1, isem[s]).wait()

  def fire_gathers(s, p):
    i0, _ = idx[s]
    for j in range(KA):
      pltpu.async_copy(h_hbm.at[i0.at[j]], rows[p].at[j], gsem[p])

  def wait_fire(s, p):
    i0, i1 = idx[s]
    for j in range(KA):
      pltpu.make_async_copy(h_hbm.at[i0.at[j]], rows[p].at[j],
                            gsem[p]).wait()
      pltpu.async_copy(rows[p].at[j], agg_sh.at[i1.at[j]], ssem[p],
                       add=True)

  def drain(s, p):
    _, i1 = idx[s]
    for j in range(KA):
      pltpu.make_async_copy(rows[p].at[j], agg_sh.at[i1.at[j]],
                            ssem[p]).wait()

  idx_start(0, 0)
  idx_wait(0, 0)
  fire_gathers(0, 0)
  idx_start(1, 1)

  def body(t, _):
    for k in (1, 2, 3, 4):
      n = 4 * t + k
      p = k % 2
      s = k % 4
      if k >= 2:
        cond_d = (4 * t + k - 2) < nblk
      else:
        cond_d = jnp.logical_and(t > 0, (4 * t + k - 2) < nblk)

      @pl.when(cond_d)
      def _():
        drain((k - 2) % 4, p)

      @pl.when(n < nblk)
      def _():
        idx_wait(s, n)
        fire_gathers(s, p)

      @pl.when(n + 1 < nblk)
      def _():
        idx_start((k + 1) % 4, n + 1)

      @pl.when(n - 1 < nblk)
      def _():
        wait_fire((k - 1) % 4, (k - 1) % 2)

    return 0

  lax.fori_loop(0, NSEC // 4, body, 0)
  @pl.when(wid < E_EXTRA)
  def _():
    drain(3, 1)
  plsc.subcore_barrier()

  oslc = pl.ds(cid * N_CONN + sid * ROWS_PER_TILE, ROWS_PER_TILE)
  pltpu.sync_copy(agg_sh.at[zslc], out_hbm.at[oslc])


def _make_agg():
  mesh = plsc.VectorSubcoreMesh(core_axis_name="c", subcore_axis_name="s",
                                num_cores=NC, num_subcores=NS)
  return pl.kernel(
      _agg_body,
      out_type=jax.ShapeDtypeStruct((NC * N_CONN, F), jnp.float32),
      mesh=mesh,
      scratch_types=(
          [pltpu.VMEM((KA, LANE, F), jnp.float32)] * 2
          + [pltpu.VMEM((KA, LANE), jnp.int32)] * 8
          + [pltpu.VMEM_SHARED((N_CONN, F), jnp.float32)]
          + [pltpu.SemaphoreType.DMA] * 8
      ),
      compiler_params=_SC_PARAMS,
  )


CP_BLOCKS = -(-C // BLOCK) + 1
CP = CP_BLOCKS * BLOCK
C_PER, C_EXTRA = divmod(CP_BLOCKS, NW)


def _gather_body(h2_hbm, xu_hbm, c0_hbm, c1_hbm, g1_hbm, g2_hbm,
                 idx0_v, idx1_v, rows1_v, rows2_v, gsem):
  cid = lax.axis_index("c")
  sid = lax.axis_index("s")
  wid = cid * NS + sid
  start = wid * C_PER + jnp.minimum(wid, C_EXTRA)
  nblk = C_PER + jnp.where(wid < C_EXTRA, 1, 0)

  def blk(b, _):
    base = (start + b) * BLOCK
    pltpu.sync_copy(c0_hbm.at[pl.ds(base, BLOCK)], idx0_v)
    pltpu.sync_copy(c1_hbm.at[pl.ds(base, BLOCK)], idx1_v)
    for j in range(KSUB):
      lslc = pl.ds(j * LANE, LANE)
      pltpu.async_copy(h2_hbm.at[idx0_v.at[lslc]], rows1_v.at[j], gsem)
      pltpu.async_copy(xu_hbm.at[idx1_v.at[lslc]], rows2_v.at[j], gsem)
    for j in range(KSUB):
      lslc = pl.ds(j * LANE, LANE)
      pltpu.make_async_copy(h2_hbm.at[idx0_v.at[lslc]], rows1_v.at[j],
                            gsem).wait()
      pltpu.make_async_copy(xu_hbm.at[idx1_v.at[lslc]], rows2_v.at[j],
                            gsem).wait()
    rowblk = (start + b) * KSUB
    pltpu.sync_copy(rows1_v, g1_hbm.at[pl.ds(rowblk, KSUB)])
    pltpu.sync_copy(rows2_v, g2_hbm.at[pl.ds(rowblk, KSUB)])
    return 0

  lax.fori_loop(0, nblk, blk, 0)


def _make_gather():
  mesh = plsc.VectorSubcoreMesh(core_axis_name="c", subcore_axis_name="s",
                                num_cores=NC, num_subcores=NS)
  return pl.kernel(
      _gather_body,
      out_type=(
          jax.ShapeDtypeStruct((CP_BLOCKS * KSUB, LANE, F), jnp.float32),
          jax.ShapeDtypeStruct((CP_BLOCKS * KSUB, LANE, F), jnp.float32),
      ),
      mesh=mesh,
      scratch_types=[
          pltpu.VMEM((BLOCK,), jnp.int32),
          pltpu.VMEM((BLOCK,), jnp.int32),
          pltpu.VMEM((KSUB, LANE, F), jnp.float32),
          pltpu.VMEM((KSUB, LANE, F), jnp.float32),
          pltpu.SemaphoreType.DMA,
      ],
      compiler_params=_SC_PARAMS,
  )


_make_agg = functools.lru_cache(None)(_make_agg)
_make_gather = functools.lru_cache(None)(_make_gather)


PACK = 128 // F


def _layer_body(agg0_ref, agg1_ref, h_ref, wr_ref, br_ref, wt_ref, out_ref):
  a = agg0_ref[...] + agg1_ref[...]
  z = (jnp.dot(a, wr_ref[...], preferred_element_type=jnp.float32)
       + jnp.dot(h_ref[...], wt_ref[...], preferred_element_type=jnp.float32)
       + br_ref[...])
  out_ref[...] = jnp.maximum(z, 0.0)


def _tc_layer(aggflat, h, wrB, brB, wtB):
  m = h.shape[0]
  blk = 1000
  grid = m // blk
  nb = m // blk
  return pl.pallas_call(
      _layer_body,
      grid=(grid,),
      in_specs=[
          pl.BlockSpec((blk, 128), lambda i: (i, 0)),
          pl.BlockSpec((blk, 128), lambda i, _nb=nb: (i + _nb, 0)),
          pl.BlockSpec((blk, 128), lambda i: (i, 0)),
          pl.BlockSpec((128, 128), lambda i: (0, 0)),
          pl.BlockSpec((1, 128), lambda i: (0, 0)),
          pl.BlockSpec((128, 128), lambda i: (0, 0)),
      ],
      out_specs=pl.BlockSpec((blk, 128), lambda i: (i, 0)),
      out_shape=jax.ShapeDtypeStruct((m, 128), jnp.float32),
  )(aggflat, aggflat, h, wrB, brB, wtB)


def _xu_body(x_ref, w_ref, b_ref, out_ref):
  out_ref[...] = (jnp.dot(x_ref[...], w_ref[...],
                          preferred_element_type=jnp.float32) + b_ref[...])


def _tc_xu(xu_in, wuB, buB):
  m = xu_in.shape[0]
  blk = m
  return pl.pallas_call(
      _xu_body,
      grid=(m // blk,),
      in_specs=[
          pl.BlockSpec((blk, 128), lambda i: (i, 0)),
          pl.BlockSpec((128, 128), lambda i: (0, 0)),
          pl.BlockSpec((1, 128), lambda i: (0, 0)),
      ],
      out_specs=pl.BlockSpec((blk, 128), lambda i: (i, 0)),
      out_shape=jax.ShapeDtypeStruct((m, 128), jnp.float32),
  )(xu_in, wuB, buB)


MLP_BLK = 256


def _mlp_body(g1_ref, g2_ref, wa_ref, wb_ref, b1_ref, w2_ref, b2_ref,
              out_ref):
  hid = (jnp.dot(g1_ref[...], wa_ref[...], preferred_element_type=jnp.float32)
         + jnp.dot(g2_ref[...], wb_ref[...],
                   preferred_element_type=jnp.float32)
         + b1_ref[...])
  hid = jnp.maximum(hid, 0.0)
  out_ref[...] = (jnp.dot(hid, w2_ref[...], preferred_element_type=jnp.float32)
                  + b2_ref[...])


def _tc_mlp(g1, g2, waB, wbB, b1B, w2B, b2):
  mp = CP // PACK
  grid = mp // MLP_BLK
  return pl.pallas_call(
      _mlp_body,
      grid=(grid,),
      in_specs=[
          pl.BlockSpec((MLP_BLK, 128), lambda i: (i, 0)),
          pl.BlockSpec((MLP_BLK, 128), lambda i: (i, 0)),
          pl.BlockSpec((128, PACK * 64), lambda i: (0, 0)),
          pl.BlockSpec((128, PACK * 64), lambda i: (0, 0)),
          pl.BlockSpec((1, PACK * 64), lambda i: (0, 0)),
          pl.BlockSpec((PACK * 64, PACK), lambda i: (0, 0)),
          pl.BlockSpec((1, PACK), lambda i: (0, 0)),
      ],
      out_specs=pl.BlockSpec((MLP_BLK, PACK), lambda i: (i, 0)),
      out_shape=jax.ShapeDtypeStruct((mp, PACK), jnp.float32),
  )(g1, g2, waB, wbB, b1B, w2B, b2)


def _padw(wT, rows=F, cols=F):
  r, c = wT.shape
  return jnp.pad(wT, ((0, rows - r), (0, cols - c)))


def _bdiag(wT):
  return jnp.kron(jnp.eye(PACK, dtype=jnp.float32), wT)


def _bdb(b, width=F):
  bp = jnp.pad(b, (0, width - b.shape[0]))
  return jnp.tile(bp, PACK)[None, :]


def kernel(x, mask, candidates, edges,
           W_rel1, b_rel1, W_root1, W_rel2, b_rel2, W_root2,
           W_u, b_u, W_f1, b_f1, W_f2, b_f2):
  e0 = edges[0].reshape(E // LANE, LANE)
  e1 = edges[1].reshape(E // LANE, LANE)
  zeros16 = jnp.zeros((N_CONN, F), jnp.float32)

  x16 = jnp.pad(x, ((0, 0), (0, F - 2)))
  xc_p = x16[:N_CONN].reshape(N_CONN // PACK, 128)
  xu_in_p = x16[N_CONN:].reshape(N_UNC // PACK, 128)

  agg1 = _make_agg()(x16, e0, e1, zeros16)
  h1_p = _tc_layer(agg1.reshape(NC * N_CONN // PACK, 128), xc_p,
                   _bdiag(_padw(W_rel1.T)), _bdb(b_rel1),
                   _bdiag(_padw(W_root1.T)))

  agg2 = _make_agg()(h1_p.reshape(N_CONN, F), e0, e1, zeros16)
  h2_p = _tc_layer(agg2.reshape(NC * N_CONN // PACK, 128), h1_p,
                   _bdiag(_padw(W_rel2.T)), _bdb(b_rel2),
                   _bdiag(_padw(W_root2.T)))

  xu_p = _tc_xu(xu_in_p, _bdiag(_padw(W_u.T)), _bdb(b_u))

  pad = CP - C
  c0 = jnp.concatenate([candidates[:, 0], jnp.zeros((pad,), jnp.int32)])
  c1 = jnp.concatenate([candidates[:, 1], jnp.zeros((pad,), jnp.int32)])
  g1, g2 = _make_gather()(h2_p.reshape(N_CONN, F), xu_p.reshape(N_UNC, F),
                          c0, c1)

  outp = _tc_mlp(g1.reshape(CP // PACK, 128), g2.reshape(CP // PACK, 128),
                 _bdiag(W_f1[:, :16].T), _bdiag(W_f1[:, 16:].T),
                 _bdb(b_f1, 64),
                 _bdiag(W_f2.T), jnp.tile(b_f2, PACK)[None, :])
  return outp.reshape(-1)[:C]
